# baseline (speedup 1.0000x reference)
import dataclasses

import jax
import jax.numpy as jnp
from jax import lax
from jax.experimental import pallas as pl
from jax.experimental.pallas import tpu as pltpu
from jax.experimental.pallas import tpu_sc as plsc

W_DIM = 64
P_DIM = 16
OUT_DIM = W_DIM + P_DIM
NC = 2
NS = 16
NW = NC * NS
LANES = 128
SB = 40


def _compiler_params():
    cp = pltpu.CompilerParams(use_tc_tiling_on_sc=False)
    if "needs_layout_passes" in pltpu.CompilerParams.__dataclass_fields__:
        cp = dataclasses.replace(cp, needs_layout_passes=False)
    return cp


def kernel(word_input, pose_input, word_table, pose_table):
    B, S = word_input.shape
    FT = OUT_DIM // 8
    BT = B // LANES
    n_blk = S // SB

    mesh = plsc.VectorSubcoreMesh(
        core_axis_name="core", subcore_axis_name="subcore"
    )

    @jax.jit
    def run(wt, pt, wi2d, pi2d):
        @pl.kernel(
            out_type=jax.ShapeDtypeStruct((S, FT, BT, 8, 128), jnp.float32),
            mesh=mesh,
            scratch_types=[
                pltpu.VMEM((S, LANES), jnp.int32),
                pltpu.VMEM((S, LANES), jnp.int32),
                pltpu.VMEM((LANES, SB), jnp.int32),
                pltpu.VMEM((100, P_DIM), jnp.float32),
                pltpu.VMEM((LANES, W_DIM), jnp.float32),
                pltpu.VMEM((LANES, W_DIM), jnp.float32),
                pltpu.VMEM((FT, 8, 128), jnp.float32),
                pltpu.VMEM((FT, 8, 128), jnp.float32),
                pltpu.SemaphoreType.DMA,
                pltpu.SemaphoreType.DMA,
                pltpu.SemaphoreType.DMA,
                pltpu.SemaphoreType.DMA,
            ],
            compiler_params=_compiler_params(),
        )
        def embed_kernel(
            wt_hbm, pt_hbm, wi_hbm, pi_hbm, o_hbm,
            itw, itp, ist, ptv, wr0, wr1, tb0, tb1,
            sg0, sg1, so0, so1,
        ):
            wid = lax.axis_index("subcore") * NC + lax.axis_index("core")
            b0 = wid * LANES
            wrv = (wr0, wr1)
            tbv = (tb0, tb1)
            sg = (sg0, sg1)
            so = (so0, so1)
            iota = lax.iota(jnp.int32, 16)

            pltpu.sync_copy(pt_hbm, ptv)

            def stage(idx_hbm, it_ref):
                @pl.loop(0, n_blk)
                def _(blk):
                    s0 = blk * SB
                    pltpu.sync_copy(
                        idx_hbm.at[pl.ds(b0, LANES), pl.ds(s0, SB)], ist
                    )

                    @pl.loop(0, SB)
                    def _(sl):
                        @pl.loop(0, LANES // 16)
                        def _(bg):
                            v = plsc.load_gather(
                                ist,
                                [bg * 16 + iota,
                                 jnp.full((16,), 0, jnp.int32) + sl],
                            )
                            it_ref[s0 + sl, pl.ds(bg * 16, 16)] = v

            stage(wi_hbm, itw)
            stage(pi_hbm, itp)

            def issue_gather(s, b):
                pltpu.make_async_copy(
                    wt_hbm.at[itw.at[s]], wrv[b], sg[b]
                ).start()

            def wait_gather(b):
                pltpu.make_async_copy(
                    wt_hbm.at[pl.ds(0, LANES)], wrv[b], sg[b]
                ).wait()

            def issue_out(s, b):
                pltpu.make_async_copy(
                    tbv[b], o_hbm.at[s, :, wid], so[b]
                ).start()

            def wait_out(b):
                pltpu.make_async_copy(
                    tbv[b], o_hbm.at[0, :, wid], so[b]
                ).wait()

            def transpose_step(s, b):
                wr = wrv[b]
                tb = tbv[b]

                @pl.loop(0, LANES // 16)
                def _(bg):
                    ridx = bg * 16 + iota
                    sl16 = pl.ds(bg * 16, 16)
                    for f in range(W_DIM):
                        v = plsc.load_gather(
                            wr, [ridx, jnp.full((16,), f, jnp.int32)]
                        )
                        tb[f // 8, f % 8, sl16] = v
                    pidx = itp[s, sl16]
                    for c in range(P_DIM):
                        v = plsc.load_gather(
                            ptv, [pidx, jnp.full((16,), c, jnp.int32)]
                        )
                        tb[8 + c // 8, c % 8, sl16] = v

            issue_gather(0, 0)

            @pl.loop(0, S // 2)
            def _(ci):
                for j in (0, 1):
                    b = j
                    s = ci * 2 + j
                    wait_gather(b)
                    if j == 0:
                        issue_gather(s + 1, 1)
                    else:
                        @pl.when(ci < S // 2 - 1)
                        def _():
                            issue_gather(s + 1, 0)

                    @pl.when(ci >= 1)
                    def _():
                        wait_out(b)

                    transpose_step(s, b)
                    issue_out(s, b)

            wait_out(0)
            wait_out(1)

        return embed_kernel(wt, pt, wi2d, pi2d)

    out5 = run(word_table, pose_table, word_input, pose_input)
    return jnp.transpose(out5, (2, 4, 0, 1, 3)).reshape(B, S, OUT_DIM)


# speedup vs baseline: 1.4664x; 1.4664x over previous
import jax
import jax.numpy as jnp
from jax import lax
from jax.experimental import pallas as pl
from jax.experimental.pallas import tpu as pltpu
from jax.experimental.pallas import tpu_sc as plsc

W_DIM = 64
P_DIM = 16
OUT_DIM = W_DIM + P_DIM
PAD_DIM = 128
NC = 2
NS = 16
NW = NC * NS
BPC = 2
GSPLIT = ((0, 128), (128, 72), (200, 128), (328, 72))


def kernel(word_input, pose_input, word_table, pose_table):
    B, S = word_input.shape
    n = B * S
    chunk = BPC * S
    b_per_w = B // NW
    n_chunks = b_per_w // BPC
    wi = word_input.reshape(n)
    pi = pose_input.reshape(n)

    mesh = plsc.VectorSubcoreMesh(
        core_axis_name="core", subcore_axis_name="subcore"
    )

    @jax.jit
    def run(wt, pt, wi, pi):
        @pl.kernel(
            out_type=jax.ShapeDtypeStruct((B, S, PAD_DIM), jnp.float32),
            mesh=mesh,
            scratch_types=[
                pltpu.VMEM((chunk,), jnp.int32),
                pltpu.VMEM((chunk,), jnp.int32),
                pltpu.VMEM((chunk,), jnp.int32),
                pltpu.VMEM((chunk,), jnp.int32),
                pltpu.VMEM((BPC, S, W_DIM), jnp.float32),
                pltpu.VMEM((BPC, S, W_DIM), jnp.float32),
                pltpu.VMEM((BPC, S, P_DIM), jnp.float32),
                pltpu.VMEM((BPC, S, P_DIM), jnp.float32),
                pltpu.SemaphoreType.DMA,
                pltpu.SemaphoreType.DMA,
                pltpu.SemaphoreType.DMA,
                pltpu.SemaphoreType.DMA,
                pltpu.SemaphoreType.DMA,
                pltpu.SemaphoreType.DMA,
            ],
            compiler_params=pltpu.CompilerParams(use_tc_tiling_on_sc=False),
        )
        def embed_kernel(
            wt_hbm, pt_hbm, wi_hbm, pi_hbm, o_hbm,
            wi0, wi1, pi0, pi1, wr0, wr1, pr0, pr1,
            si0, si1, sg0, sg1, so0, so1,
        ):
            wid = lax.axis_index("subcore") * NC + lax.axis_index("core")
            base = wid * b_per_w * S
            bbase = wid * b_per_w
            wiv = (wi0, wi1)
            piv = (pi0, pi1)
            wrv = (wr0, wr1)
            prv = (pr0, pr1)
            si = (si0, si1)
            sg = (sg0, sg1)
            so = (so0, so1)

            def issue_idx(c, b):
                off = base + c * chunk
                pltpu.make_async_copy(
                    wi_hbm.at[pl.ds(off, chunk)], wiv[b], si[b]
                ).start()
                pltpu.make_async_copy(
                    pi_hbm.at[pl.ds(off, chunk)], piv[b], si[b]
                ).start()

            def wait_idx(b):
                pltpu.make_async_copy(
                    wi_hbm.at[pl.ds(base, chunk)], wiv[b], si[b]
                ).wait()
                pltpu.make_async_copy(
                    pi_hbm.at[pl.ds(base, chunk)], piv[b], si[b]
                ).wait()

            def issue_gathers(b):
                for g, (off, ln) in enumerate(GSPLIT):
                    bl = off // S
                    so_ = off % S
                    pltpu.make_async_copy(
                        wt_hbm.at[wiv[b].at[pl.ds(off, ln)]],
                        wrv[b].at[bl, pl.ds(so_, ln)],
                        sg[b],
                    ).start()
                    pltpu.make_async_copy(
                        pt_hbm.at[piv[b].at[pl.ds(off, ln)]],
                        prv[b].at[bl, pl.ds(so_, ln)],
                        sg[b],
                    ).start()

            def wait_gathers(b):
                pltpu.make_async_copy(
                    o_hbm.at[pl.ds(bbase, BPC), :, 0:W_DIM], wrv[b], sg[b]
                ).wait()
                pltpu.make_async_copy(
                    o_hbm.at[pl.ds(bbase, BPC), :, W_DIM:OUT_DIM],
                    prv[b],
                    sg[b],
                ).wait()

            def issue_out(c, b):
                bo = bbase + c * BPC
                pltpu.make_async_copy(
                    wrv[b], o_hbm.at[pl.ds(bo, BPC), :, 0:W_DIM], so[b]
                ).start()
                pltpu.make_async_copy(
                    prv[b], o_hbm.at[pl.ds(bo, BPC), :, W_DIM:OUT_DIM], so[b]
                ).start()

            def wait_out(b):
                pltpu.make_async_copy(
                    wrv[b], o_hbm.at[pl.ds(bbase, BPC), :, 0:W_DIM], so[b]
                ).wait()
                pltpu.make_async_copy(
                    prv[b], o_hbm.at[pl.ds(bbase, BPC), :, W_DIM:OUT_DIM],
                    so[b],
                ).wait()

            issue_idx(0, 0)

            @pl.loop(0, n_chunks // 2)
            def _(ci):
                for j in (0, 1):
                    b = j
                    c = ci * 2 + j
                    wait_idx(b)

                    @pl.when(ci >= 1)
                    def _():
                        wait_out(b)

                    issue_gathers(b)
                    if j == 0:
                        @pl.when(ci >= 1)
                        def _():
                            wait_gathers(1)
                            issue_out(c - 1, 1)
                        issue_idx(c + 1, 1)
                    else:
                        wait_gathers(0)
                        issue_out(c - 1, 0)

                        @pl.when(ci < n_chunks // 2 - 1)
                        def _():
                            issue_idx(c + 1, 0)

            wait_gathers(1)
            issue_out(n_chunks - 1, 1)
            wait_out(0)
            wait_out(1)

        return embed_kernel(wt, pt, wi, pi)

    out = run(word_table, pose_table, wi, pi)
    return out[:, :, :OUT_DIM]


# speedup vs baseline: 1.7683x; 1.2059x over previous
import dataclasses

import jax
import jax.numpy as jnp
from jax import lax
from jax.experimental import pallas as pl
from jax.experimental.pallas import tpu as pltpu
from jax.experimental.pallas import tpu_sc as plsc


def _compiler_params():
    cp = pltpu.CompilerParams(use_tc_tiling_on_sc=False)
    if "needs_layout_passes" in pltpu.CompilerParams.__dataclass_fields__:
        cp = dataclasses.replace(cp, needs_layout_passes=False)
    return cp

W_DIM = 64
P_DIM = 16
OUT_DIM = W_DIM + P_DIM
PAD_DIM = 128
NC = 2
NS = 16
NW = NC * NS
BPC = 2
GSPLIT = ((0, 128), (128, 72), (200, 128), (328, 72))


def kernel(word_input, pose_input, word_table, pose_table):
    B, S = word_input.shape
    n = B * S
    chunk = BPC * S
    b_per_w = B // NW
    n_chunks = b_per_w // BPC
    wi = word_input.reshape(n)
    pi = pose_input.reshape(n)

    mesh = plsc.VectorSubcoreMesh(
        core_axis_name="core", subcore_axis_name="subcore"
    )

    @jax.jit
    def run(wt, pt, wi, pi):
        @pl.kernel(
            out_type=jax.ShapeDtypeStruct((B, S, PAD_DIM), jnp.float32),
            mesh=mesh,
            scratch_types=[
                pltpu.VMEM((chunk,), jnp.int32),
                pltpu.VMEM((chunk,), jnp.int32),
                pltpu.VMEM((chunk,), jnp.int32),
                pltpu.VMEM((chunk,), jnp.int32),
                pltpu.VMEM((BPC, S, W_DIM), jnp.float32),
                pltpu.VMEM((BPC, S, W_DIM), jnp.float32),
                pltpu.VMEM((BPC, S, P_DIM), jnp.float32),
                pltpu.VMEM((BPC, S, P_DIM), jnp.float32),
                pltpu.VMEM((100, P_DIM), jnp.float32),
                pltpu.SemaphoreType.DMA,
                pltpu.SemaphoreType.DMA,
                pltpu.SemaphoreType.DMA,
                pltpu.SemaphoreType.DMA,
                pltpu.SemaphoreType.DMA,
                pltpu.SemaphoreType.DMA,
            ],
            compiler_params=_compiler_params(),
        )
        def embed_kernel(
            wt_hbm, pt_hbm, wi_hbm, pi_hbm, o_hbm,
            wi0, wi1, pi0, pi1, wr0, wr1, pr0, pr1, ptv,
            si0, si1, sg0, sg1, so0, so1,
        ):
            wid = lax.axis_index("subcore") * NC + lax.axis_index("core")
            base = wid * b_per_w * S
            bbase = wid * b_per_w
            wiv = (wi0, wi1)
            piv = (pi0, pi1)
            wrv = (wr0, wr1)
            prv = (pr0, pr1)
            si = (si0, si1)
            sg = (sg0, sg1)
            so = (so0, so1)

            def issue_idx(c, b):
                off = base + c * chunk
                pltpu.make_async_copy(
                    wi_hbm.at[pl.ds(off, chunk)], wiv[b], si[b]
                ).start()
                pltpu.make_async_copy(
                    pi_hbm.at[pl.ds(off, chunk)], piv[b], si[b]
                ).start()

            def wait_idx(b):
                pltpu.make_async_copy(
                    wi_hbm.at[pl.ds(base, chunk)], wiv[b], si[b]
                ).wait()
                pltpu.make_async_copy(
                    pi_hbm.at[pl.ds(base, chunk)], piv[b], si[b]
                ).wait()

            def issue_gathers(b):
                for g, (off, ln) in enumerate(GSPLIT):
                    bl = off // S
                    so_ = off % S
                    pltpu.make_async_copy(
                        wt_hbm.at[wiv[b].at[pl.ds(off, ln)]],
                        wrv[b].at[bl, pl.ds(so_, ln)],
                        sg[b],
                    ).start()

            def wait_gathers(b):
                pltpu.make_async_copy(
                    o_hbm.at[pl.ds(bbase, BPC), :, 0:W_DIM], wrv[b], sg[b]
                ).wait()

            def pose_fill(b):
                iota = lax.iota(jnp.int32, 16)

                @pl.loop(0, chunk // 16)
                def _(bg):
                    r = bg * 16 + iota
                    blv = r // S
                    sv = r - blv * S
                    sl16 = pl.ds(bg * 16, 16)
                    pidx = piv[b][sl16]
                    for c in range(P_DIM):
                        v = plsc.load_gather(
                            ptv, [pidx, jnp.full((16,), c, jnp.int32)]
                        )
                        plsc.store_scatter(
                            prv[b], [blv, sv, jnp.full((16,), c, jnp.int32)], v
                        )

            def issue_out(c, b):
                bo = bbase + c * BPC
                pltpu.make_async_copy(
                    wrv[b], o_hbm.at[pl.ds(bo, BPC), :, 0:W_DIM], so[b]
                ).start()
                pltpu.make_async_copy(
                    prv[b], o_hbm.at[pl.ds(bo, BPC), :, W_DIM:OUT_DIM], so[b]
                ).start()

            def wait_out(b):
                pltpu.make_async_copy(
                    wrv[b], o_hbm.at[pl.ds(bbase, BPC), :, 0:W_DIM], so[b]
                ).wait()
                pltpu.make_async_copy(
                    prv[b], o_hbm.at[pl.ds(bbase, BPC), :, W_DIM:OUT_DIM],
                    so[b],
                ).wait()

            pltpu.sync_copy(pt_hbm, ptv)
            issue_idx(0, 0)

            @pl.loop(0, n_chunks // 2)
            def _(ci):
                for j in (0, 1):
                    b = j
                    c = ci * 2 + j
                    wait_idx(b)

                    @pl.when(ci >= 1)
                    def _():
                        wait_out(b)

                    issue_gathers(b)
                    pose_fill(b)
                    if j == 0:
                        @pl.when(ci >= 1)
                        def _():
                            wait_gathers(1)
                            issue_out(c - 1, 1)
                        issue_idx(c + 1, 1)
                    else:
                        wait_gathers(0)
                        issue_out(c - 1, 0)

                        @pl.when(ci < n_chunks // 2 - 1)
                        def _():
                            issue_idx(c + 1, 0)

            wait_gathers(1)
            issue_out(n_chunks - 1, 1)
            wait_out(0)
            wait_out(1)

        return embed_kernel(wt, pt, wi, pi)

    out = run(word_table, pose_table, wi, pi)
    return out[:, :, :OUT_DIM]
